# baseline (speedup 1.0000x reference)
import dataclasses
import functools

import jax
import jax.numpy as jnp
from jax import lax
from jax.experimental import pallas as pl
from jax.experimental.pallas import tpu as pltpu
from jax.experimental.pallas import tpu_sc as plsc

NC = 2
NS = 16
L = 16
NW = NC * NS
B = 16384
D = 32
BPW = B // NW
CH = 128
NCHK = BPW // CH
CHG = CH // L
NG = BPW // L
NQ = 4
SPQ = CH // NQ

_mesh = plsc.VectorSubcoreMesh(core_axis_name="c", subcore_axis_name="s")

_cp = pltpu.CompilerParams()
if "needs_layout_passes" in pltpu.CompilerParams.__dataclass_fields__:
    _cp = dataclasses.replace(_cp, needs_layout_passes=False)


def _gmf_body(uid_hbm, iid_hbm, utab_hbm, itab_hbm, w_hbm, b_hbm, out_hbm,
              usm, ism, uidx_v, iidx_v, ub0, ub1, ib0, ib1, wsc_v, b_v, o_v,
              idsem, *sems):
    wid = lax.axis_index("s") * NC + lax.axis_index("c")
    base = wid * BPW
    ubufs = (ub0, ub1)
    ibufs = (ib0, ib1)
    usems = ((sems[0], sems[1], sems[2], sems[3]),
             (sems[8], sems[9], sems[10], sems[11]))
    isems = ((sems[4], sems[5], sems[6], sems[7]),
             (sems[12], sems[13], sems[14], sems[15]))

    sid = lax.axis_index("s")
    pltpu.async_copy(uid_hbm.at[pl.ds(base, BPW)], uidx_v.at[sid], idsem).wait()
    pltpu.async_copy(iid_hbm.at[pl.ds(base, BPW)], iidx_v.at[sid], idsem).wait()
    pltpu.sync_copy(uidx_v.at[sid], usm)
    pltpu.sync_copy(iidx_v.at[sid], ism)
    pltpu.sync_copy(w_hbm, wsc_v)
    pltpu.sync_copy(b_hbm, b_v)

    def fire(c):
        p = c % 2
        ubp, ibp = ubufs[p], ibufs[p]
        for q in range(NQ):
            usem, isem = usems[p][q], isems[p][q]

            @plsc.parallel_loop(0, SPQ, unroll=8)
            def _(r):
                rr = r * NQ + q
                j = c * CH + rr
                pltpu.async_copy(utab_hbm.at[usm[j]], ubp.at[rr], usem)
                pltpu.async_copy(itab_hbm.at[ism[j]], ibp.at[rr], isem)

    def drain(c):
        p = c % 2
        for q in range(NQ):
            pltpu.make_async_copy(
                utab_hbm.at[pl.ds(0, SPQ)], ubufs[p].at[pl.ds(0, SPQ)],
                usems[p][q]).wait()
            pltpu.make_async_copy(
                itab_hbm.at[pl.ds(0, SPQ)], ibufs[p].at[pl.ds(0, SPQ)],
                isems[p][q]).wait()

    fire(0)
    for c in range(NCHK):
        if c + 1 < NCHK:
            fire(c + 1)
        drain(c)
        ubp, ibp = ubufs[c % 2], ibufs[c % 2]

        @pl.loop(0, CHG)
        def _(k):
            g = c * CHG + k
            rows = k * L + lax.iota(jnp.int32, L)
            acc = b_v[...]
            for d in range(D):
                didx = jnp.full((L,), d, jnp.int32)
                uv = plsc.load_gather(ubp, [rows, didx])
                iv = plsc.load_gather(ibp, [rows, didx])
                wv = wsc_v.at[(d, pl.ds(0, L))][...]
                acc = acc + uv * iv * wv
            o_v.at[(g, pl.ds(0, L))][...] = 1.0 / (1.0 + jnp.exp(-acc))

    pltpu.sync_copy(o_v, out_hbm.at[wid])


@functools.partial(
    pl.kernel,
    out_type=jax.ShapeDtypeStruct((NW, NG, L), jnp.float32),
    mesh=_mesh,
    scratch_types=[
        pltpu.SMEM((BPW,), jnp.int32),
        pltpu.SMEM((BPW,), jnp.int32),
        pltpu.VMEM_SHARED((NS, BPW), jnp.int32),
        pltpu.VMEM_SHARED((NS, BPW), jnp.int32),
        pltpu.VMEM((CH, D), jnp.float32),
        pltpu.VMEM((CH, D), jnp.float32),
        pltpu.VMEM((CH, D), jnp.float32),
        pltpu.VMEM((CH, D), jnp.float32),
        pltpu.VMEM((D, L), jnp.float32),
        pltpu.VMEM((L,), jnp.float32),
        pltpu.VMEM((NG, L), jnp.float32),
        pltpu.SemaphoreType.DMA,
    ] + [pltpu.SemaphoreType.DMA] * 16,
    compiler_params=_cp,
)
def _gmf_sc(*args):
    _gmf_body(*args)


@jax.jit
def kernel(user_ids, item_ids, user_table, item_table, W, b):
    uid = user_ids.astype(jnp.int32)
    iid = item_ids.astype(jnp.int32)
    w_bcast = jnp.broadcast_to(W.reshape(D, 1), (D, L)).astype(jnp.float32)
    b16 = jnp.full((L,), b[0], dtype=jnp.float32)
    out3 = _gmf_sc(uid, iid, user_table, item_table, w_bcast, b16)
    return out3.reshape(B)
